# baseline (speedup 1.0000x reference)
import functools

import jax
import jax.numpy as jnp
from jax.experimental import pallas as pl
from jax.experimental.pallas import tpu as pltpu

_D_MODEL = 1024
_NUM_EXPERTS = 8
_HIDDEN = 2048
_TOKENS = 2048

_TILE_T = 256


def _routing_kernel(x_ref, wr_ref, combine_ref, counts_ref):
    logits = jax.lax.dot_general(
        x_ref[...].astype(jnp.bfloat16), wr_ref[...].astype(jnp.bfloat16),
        (((1,), (0,)), ((), ())),
        preferred_element_type=jnp.float32)
    t, e = logits.shape
    col = jax.lax.broadcasted_iota(jnp.int32, (t, e), 1)

    v1 = jnp.max(logits, axis=1, keepdims=True)
    i1 = jnp.min(jnp.where(logits == v1, col, e), axis=1, keepdims=True)
    masked = jnp.where(col == i1, -jnp.inf, logits)
    v2 = jnp.max(masked, axis=1, keepdims=True)
    i2 = jnp.min(jnp.where(masked == v2, col, e), axis=1, keepdims=True)

    e2 = jnp.exp(v2 - v1)
    w1 = 1.0 / (1.0 + e2)
    w2 = 1.0 - w1

    sel1 = (col == i1)
    sel2 = (col == i2)
    combine_ref[...] = jnp.where(sel1, w1, 0.0) + jnp.where(sel2, w2, 0.0)
    counts_ref[...] = jnp.sum(
        jnp.where(sel1 | sel2, 1.0, 0.0), axis=0, keepdims=True)


def _expert_kernel(x_ref, cmb_ref, win_ref, wout_ref, out_ref, acc_ref):
    e = pl.program_id(0)
    m = pl.program_id(1)
    n_e = pl.num_programs(0)

    xb = x_ref[...]
    h = jax.lax.dot_general(
        xb, win_ref[0], (((1,), (0,)), ((), ())),
        preferred_element_type=jnp.float32)
    h = h * jax.nn.sigmoid(h)
    y = jax.lax.dot_general(
        h.astype(jnp.bfloat16), wout_ref[0], (((1,), (0,)), ((), ())),
        preferred_element_type=jnp.float32)
    cmb = cmb_ref[...]
    ecol = jax.lax.broadcasted_iota(jnp.int32, cmb.shape, 1)
    w = jnp.sum(jnp.where(ecol == e, cmb, 0.0), axis=1, keepdims=True)
    contrib = w * y

    sl = pl.ds(m * _TILE_T, _TILE_T)

    @pl.when(e == 0)
    def _():
        acc_ref[sl, :] = contrib

    @pl.when(e != 0)
    def _():
        acc_ref[sl, :] = acc_ref[sl, :] + contrib

    @pl.when(e == n_e - 1)
    def _():
        out_ref[...] = acc_ref[sl, :]


def _moe_fwd(x, w_router, w_in, w_out):
    t, d = x.shape
    n_exp = w_in.shape[0]
    hid = w_in.shape[2]

    combine, counts = pl.pallas_call(
        _routing_kernel,
        out_shape=(
            jax.ShapeDtypeStruct((t, n_exp), jnp.float32),
            jax.ShapeDtypeStruct((1, n_exp), jnp.float32),
        ),
    )(x, w_router)

    xb = x.astype(jnp.bfloat16)
    winb = w_in.astype(jnp.bfloat16)
    woutb = w_out.astype(jnp.bfloat16)

    n_m = t // _TILE_T
    out = pl.pallas_call(
        _expert_kernel,
        grid=(n_exp, n_m),
        in_specs=[
            pl.BlockSpec((_TILE_T, d), lambda e, m: (m, 0)),
            pl.BlockSpec((_TILE_T, _NUM_EXPERTS), lambda e, m: (m, 0)),
            pl.BlockSpec((1, d, hid), lambda e, m: (e, 0, 0)),
            pl.BlockSpec((1, hid, d), lambda e, m: (e, 0, 0)),
        ],
        out_specs=pl.BlockSpec((_TILE_T, d), lambda e, m: (m, 0)),
        out_shape=jax.ShapeDtypeStruct((t, d), jnp.float32),
        scratch_shapes=[pltpu.VMEM((t, d), jnp.float32)],
        compiler_params=pltpu.CompilerParams(
            dimension_semantics=("arbitrary", "arbitrary"),
        ),
    )(xb, combine, winb, woutb)

    return out, counts.reshape(n_exp)


def kernel(x, w_router, w_in, w_out):
    return _moe_fwd(x, w_router, w_in, w_out)
